# baseline (speedup 1.0000x reference)
import functools

import jax
import jax.numpy as jnp
from jax import lax
from jax.experimental import pallas as pl
from jax.experimental.pallas import tpu as pltpu
from jax.experimental.pallas import tpu_sc as plsc

N, E, D, DE, H = 10000, 320000, 128, 4, 128
NPAD = 10240
NC, NS = 2, 16
NW = NC * NS
C = 80
ROWS_PER_TILE = NPAD // NS
HALVES = ((0, 179200), (179200, 140800))

f32 = jnp.float32
i32 = jnp.int32
MESH = dict(core_axis_name="c", subcore_axis_name="s",
            num_cores=NC, num_subcores=NS)


def _prep_body(h_ref, w1a_ref, w1b_ref, ta_ref, tb_ref):
    hb = h_ref[...]
    ta_ref[...] = jnp.dot(hb, w1a_ref[...], preferred_element_type=f32)
    tb_ref[...] = jnp.dot(hb, w1b_ref[...], preferred_element_type=f32)


def _prep_tables(h, w1a, w1b):
    bn = 1000
    return pl.pallas_call(
        _prep_body,
        grid=(N // bn,),
        in_specs=[
            pl.BlockSpec((bn, 128), lambda i: (i, 0)),
            pl.BlockSpec((128, 128), lambda i: (0, 0)),
            pl.BlockSpec((128, 128), lambda i: (0, 0)),
        ],
        out_specs=[
            pl.BlockSpec((bn, 128), lambda i: (i, 0)),
            pl.BlockSpec((bn, 128), lambda i: (i, 0)),
        ],
        out_shape=[
            jax.ShapeDtypeStruct((N, 128), f32),
            jax.ShapeDtypeStruct((N, 128), f32),
        ],
    )(h, w1a, w1b)


@functools.cache
def _gather_pre(e0, ne):
    PER_W = ne // NW
    CH = PER_W // C

    def body(ta, tb, row, col, out, idxr, idxc, bufr, bufc,
             semr0, semr1, semc0, semc1):
        c = lax.axis_index("c")
        s = lax.axis_index("s")
        wid = s * NC + c
        gbase = e0 + wid * PER_W
        obase = wid * PER_W
        pltpu.sync_copy(row.at[pl.ds(gbase, PER_W)], idxr)
        pltpu.sync_copy(col.at[pl.ds(gbase, PER_W)], idxc)
        semr = (semr0, semr1)
        semc = (semc0, semc1)

        def start(k, p):
            pltpu.async_copy(ta.at[idxr.at[pl.ds(k * C, C)]], bufr.at[p],
                             semr[p])
            pltpu.async_copy(tb.at[idxc.at[pl.ds(k * C, C)]], bufc.at[p],
                             semc[p])

        def bodyc(k, p):
            pltpu.make_async_copy(ta.at[idxr.at[pl.ds(0, C)]], bufr.at[p],
                                  semr[p]).wait()
            pltpu.make_async_copy(tb.at[idxc.at[pl.ds(0, C)]], bufc.at[p],
                                  semc[p]).wait()

            def rowfn(i, carry2):
                for j in range(8):
                    sl = pl.ds(j * 16, 16)
                    bufr[p, i, sl] = bufr[p, i, sl] + bufc[p, i, sl]
                return carry2

            lax.fori_loop(0, C, rowfn, 0)
            pltpu.sync_copy(bufr.at[p], out.at[pl.ds(obase + k * C, C)])

        start(0, 0)
        start(1, 1)

        def chunk2(k2, carry):
            for p in range(2):
                k = k2 * 2 + p
                bodyc(k, p)

                @pl.when(k + 2 < CH)
                def _():
                    start(k + 2, p)
            return carry

        lax.fori_loop(0, CH // 2, chunk2, 0)

        @pl.when((CH % 2) == 1)
        def _():
            bodyc(CH - 1, 0)

    return pl.kernel(
        body,
        out_type=jax.ShapeDtypeStruct((ne, 128), f32),
        mesh=plsc.VectorSubcoreMesh(**MESH),
        scratch_types=[
            pltpu.VMEM((PER_W,), i32),
            pltpu.VMEM((PER_W,), i32),
            pltpu.VMEM((2, C, 128), f32),
            pltpu.VMEM((2, C, 128), f32),
            pltpu.SemaphoreType.DMA,
            pltpu.SemaphoreType.DMA,
            pltpu.SemaphoreType.DMA,
            pltpu.SemaphoreType.DMA,
        ],
    )


@functools.cache
def _coord_feats(e0, ne):
    PER_W = ne // NW
    CH = PER_W // C

    def body(c16, row, col, out, idxr, idxc, bufr, bufc,
             semr0, semr1, semc0, semc1):
        c = lax.axis_index("c")
        s = lax.axis_index("s")
        wid = s * NC + c
        gbase = e0 + wid * PER_W
        obase = wid * PER_W
        pltpu.sync_copy(row.at[pl.ds(gbase, PER_W)], idxr)
        pltpu.sync_copy(col.at[pl.ds(gbase, PER_W)], idxc)
        semr = (semr0, semr1)
        semc = (semc0, semc1)

        def start(k, p):
            pltpu.async_copy(c16.at[idxr.at[pl.ds(k * C, C)]], bufr.at[p],
                             semr[p])
            pltpu.async_copy(c16.at[idxc.at[pl.ds(k * C, C)]], bufc.at[p],
                             semc[p])

        def bodyc(k, p):
            pltpu.make_async_copy(c16.at[idxr.at[pl.ds(0, C)]], bufr.at[p],
                                  semr[p]).wait()
            pltpu.make_async_copy(c16.at[idxc.at[pl.ds(0, C)]], bufc.at[p],
                                  semc[p]).wait()

            def rowfn(i, carry2):
                bufr[p, i, :] = bufr[p, i, :] - bufc[p, i, :]
                return carry2

            lax.fori_loop(0, C, rowfn, 0)
            pltpu.sync_copy(bufr.at[p], out.at[pl.ds(obase + k * C, C)])

        start(0, 0)
        start(1, 1)

        def chunk2(k2, carry):
            for p in range(2):
                k = k2 * 2 + p
                bodyc(k, p)

                @pl.when(k + 2 < CH)
                def _():
                    start(k + 2, p)
            return carry

        lax.fori_loop(0, CH // 2, chunk2, 0)

        @pl.when((CH % 2) == 1)
        def _():
            bodyc(CH - 1, 0)

    return pl.kernel(
        body,
        out_type=jax.ShapeDtypeStruct((ne, 16), f32),
        mesh=plsc.VectorSubcoreMesh(**MESH),
        compiler_params=pltpu.CompilerParams(use_tc_tiling_on_sc=False,
                                             needs_layout_passes=False),
        scratch_types=[
            pltpu.VMEM((PER_W,), i32),
            pltpu.VMEM((PER_W,), i32),
            pltpu.VMEM((2, C, 16), f32),
            pltpu.VMEM((2, C, 16), f32),
            pltpu.SemaphoreType.DMA,
            pltpu.SemaphoreType.DMA,
            pltpu.SemaphoreType.DMA,
            pltpu.SemaphoreType.DMA,
        ],
    )


def _edge_body(pre_ref, dr_ref, ea_ref, wr_ref, wea_ref, be1_ref,
               we2_ref, be2_ref, wc1_ref, bc1_ref, wc2_ref, bc2_ref,
               m_ref, t_ref):
    be = m_ref.shape[0]
    dr = dr_ref[...]
    rad = jnp.sum(dr * dr, axis=1, keepdims=True)
    x1 = (pre_ref[...] + rad * wr_ref[...] +
          jnp.dot(ea_ref[...], wea_ref[...], preferred_element_type=f32) +
          be1_ref[...])
    x1 = jnp.maximum(x1, 0.0)
    m = jnp.maximum(
        jnp.dot(x1, we2_ref[...], preferred_element_type=f32) + be2_ref[...],
        0.0)
    m_ref[...] = m
    cfh = jnp.maximum(
        jnp.dot(m, wc1_ref[...], preferred_element_type=f32) + bc1_ref[...],
        0.0)
    cf = jnp.dot(cfh, wc2_ref[...], preferred_element_type=f32) + bc2_ref[...]
    t = dr * cf
    iot = lax.broadcasted_iota(i32, t.shape, 1)
    t16 = jnp.where(iot == 3, 1.0, t)
    t_ref[...] = jnp.concatenate([t16, jnp.zeros((be, 112), f32)], axis=1)


def _edge_mlp(e0, ne, pre, dr, edge_attr,
              wr, wea, be1, we2, be2, wc1, bc1, wc2, bc2):
    be = 1600
    off = e0 // be
    wfull = lambda shape: pl.BlockSpec(shape, lambda i: (0, 0))
    return pl.pallas_call(
        _edge_body,
        grid=(ne // be,),
        in_specs=[
            pl.BlockSpec((be, 128), lambda i: (i, 0)),
            pl.BlockSpec((be, 16), lambda i: (i, 0)),
            pl.BlockSpec((be, DE), lambda i: (i + off, 0)),
            wfull((1, 128)), wfull((DE, 128)), wfull((1, 128)),
            wfull((128, 128)), wfull((1, 128)),
            wfull((128, 128)), wfull((1, 128)),
            wfull((128, 1)), wfull((1, 1)),
        ],
        out_specs=[
            pl.BlockSpec((be, 128), lambda i: (i, 0)),
            pl.BlockSpec((be, 128), lambda i: (i, 0)),
        ],
        out_shape=[
            jax.ShapeDtypeStruct((ne, 128), f32),
            jax.ShapeDtypeStruct((ne, 128), f32),
        ],
    )(pre, dr, edge_attr, wr, wea, be1, we2, be2, wc1, bc1, wc2, bc2)


def _make_scatter(e0, ne):
    PER_W = ne // NW
    CH = PER_W // C

    def body(val, row, z128, acc_out, idxv, vbuf, accsh, sem0, sem1):
        c = lax.axis_index("c")
        s = lax.axis_index("s")
        wid = s * NC + c
        ibase = e0 + wid * PER_W
        vbase = wid * PER_W
        rsl = pl.ds(s * ROWS_PER_TILE, ROWS_PER_TILE)
        pltpu.sync_copy(z128.at[rsl], accsh.at[rsl])
        plsc.subcore_barrier()
        sems = (sem0, sem1)

        def start(k, p):
            pltpu.sync_copy(row.at[pl.ds(ibase + k * C, C)], idxv.at[p])
            pltpu.async_copy(val.at[pl.ds(vbase + k * C, C)], vbuf.at[p],
                             sems[p])

        def bodyc(p):
            pltpu.make_async_copy(val.at[pl.ds(0, C)], vbuf.at[p],
                                  sems[p]).wait()
            pltpu.sync_copy(vbuf.at[p], accsh.at[idxv.at[p]], add=True)

        start(0, 0)
        start(1, 1)

        def chunk2(k2, carry):
            for p in range(2):
                k = k2 * 2 + p
                bodyc(p)

                @pl.when(k + 2 < CH)
                def _():
                    start(k + 2, p)
            return carry

        lax.fori_loop(0, CH // 2, chunk2, 0)

        @pl.when((CH % 2) == 1)
        def _():
            bodyc(0)

        plsc.subcore_barrier()
        pltpu.sync_copy(accsh.at[rsl], acc_out.at[c].at[rsl])

    return pl.kernel(
        body,
        out_type=jax.ShapeDtypeStruct((NC, NPAD, 128), f32),
        mesh=plsc.VectorSubcoreMesh(**MESH),
        scratch_types=[
            pltpu.VMEM((2, C), i32),
            pltpu.VMEM((2, C, 128), f32),
            pltpu.VMEM_SHARED((NPAD, 128), f32),
            pltpu.SemaphoreType.DMA,
            pltpu.SemaphoreType.DMA,
        ],
    )


@functools.cache
def _scatter_m(e0, ne):
    return _make_scatter(e0, ne)


@functools.cache
def _scatter_t(e0, ne):
    return _make_scatter(e0, ne)


def _node_body(h_ref, a00_ref, a01_ref, a10_ref, a11_ref,
               t00_ref, t01_ref, t10_ref, t11_ref, coord_ref,
               wn1a_ref, wn1b_ref, bn1_ref, wn2_ref, bn2_ref,
               hout_ref, cout_ref):
    agg = a00_ref[0] + a01_ref[0] + a10_ref[0] + a11_ref[0]
    u = jnp.maximum(
        jnp.dot(h_ref[...], wn1a_ref[...], preferred_element_type=f32) +
        jnp.dot(agg, wn1b_ref[...], preferred_element_type=f32) +
        bn1_ref[...], 0.0)
    hout_ref[...] = (jnp.dot(u, wn2_ref[...], preferred_element_type=f32) +
                     bn2_ref[...])
    t = t00_ref[0] + t01_ref[0] + t10_ref[0] + t11_ref[0]
    s3 = t[:, 0:3]
    cnt = t[:, 3:4]
    cout_ref[...] = coord_ref[...] + s3 / jnp.maximum(cnt, 1.0)


def _node_model(h, agg0, agg1, t0, t1, coord, wn1a, wn1b, bn1, wn2, bn2):
    bn = 1000
    wfull = lambda shape: pl.BlockSpec(shape, lambda i: (0, 0))
    p0 = pl.BlockSpec((1, bn, 128), lambda i: (0, i, 0))
    p1 = pl.BlockSpec((1, bn, 128), lambda i: (1, i, 0))
    return pl.pallas_call(
        _node_body,
        grid=(N // bn,),
        in_specs=[
            pl.BlockSpec((bn, 128), lambda i: (i, 0)),
            p0, p1, p0, p1, p0, p1, p0, p1,
            pl.BlockSpec((bn, 3), lambda i: (i, 0)),
            wfull((128, 128)), wfull((128, 128)), wfull((1, 128)),
            wfull((128, 128)), wfull((1, 128)),
        ],
        out_specs=[
            pl.BlockSpec((bn, 128), lambda i: (i, 0)),
            pl.BlockSpec((bn, 3), lambda i: (i, 0)),
        ],
        out_shape=[
            jax.ShapeDtypeStruct((N, 128), f32),
            jax.ShapeDtypeStruct((N, 3), f32),
        ],
    )(h, agg0, agg0, agg1, agg1, t0, t0, t1, t1, coord,
      wn1a, wn1b, bn1, wn2, bn2)


def kernel(h, edge_index, coord, edge_attr,
           W_e1, b_e1, W_e2, b_e2,
           W_n1, b_n1, W_n2, b_n2,
           W_c1, b_c1, W_c2, b_c2):
    row = edge_index[0]
    col = edge_index[1]
    c16 = jnp.pad(coord, ((0, 0), (0, 13)))
    w1a = W_e1[0:D]
    w1b = W_e1[D:2 * D]
    wr = W_e1[2 * D:2 * D + 1]
    wea = W_e1[2 * D + 1:]
    ew = (wr, wea, b_e1.reshape(1, H), W_e2, b_e2.reshape(1, H),
          W_c1, b_c1.reshape(1, H), W_c2, b_c2.reshape(1, 1))
    ta, tb = _prep_tables(h, w1a, w1b)
    z128 = jnp.zeros((NPAD, 128), f32)

    (a0, na), (a1, nb) = HALVES
    dr0 = _coord_feats(a0, na)(c16, row, col)
    pre0 = _gather_pre(a0, na)(ta, tb, row, col)
    m0, t0 = _edge_mlp(a0, na, pre0, dr0, edge_attr, *ew)
    dr1 = _coord_feats(a1, nb)(c16, row, col)
    pre1 = _gather_pre(a1, nb)(ta, tb, row, col)
    m1, t1 = _edge_mlp(a1, nb, pre1, dr1, edge_attr, *ew)

    agg0 = _scatter_m(a0, na)(m0, row, z128)
    t_p0 = _scatter_t(a0, na)(t0, row, z128)
    agg1 = _scatter_m(a1, nb)(m1, row, z128)
    t_p1 = _scatter_t(a1, nb)(t1, row, z128)

    m_ij = jnp.concatenate([m0, m1], axis=0)
    h_out, coord_out = _node_model(h, agg0, agg1, t_p0, t_p1, coord,
                                   W_n1[0:D], W_n1[D:], b_n1.reshape(1, H),
                                   W_n2, b_n2.reshape(1, H))
    return (h_out, coord_out, m_ij)


# speedup vs baseline: 1.0163x; 1.0163x over previous
import functools

import jax
import jax.numpy as jnp
from jax import lax
from jax.experimental import pallas as pl
from jax.experimental.pallas import tpu as pltpu
from jax.experimental.pallas import tpu_sc as plsc

N, E, D, DE, H = 10000, 320000, 128, 4, 128
NPAD = 10240
NC, NS = 2, 16
NW = NC * NS
C = 80
ROWS_PER_TILE = NPAD // NS
HALVES = ((0, 179200), (179200, 140800))

f32 = jnp.float32
i32 = jnp.int32
MESH = dict(core_axis_name="c", subcore_axis_name="s",
            num_cores=NC, num_subcores=NS)


def _prep_body(h_ref, w1a_ref, w1b_ref, ta_ref, tb_ref):
    hb = h_ref[...]
    ta_ref[...] = jnp.dot(hb, w1a_ref[...], preferred_element_type=f32)
    tb_ref[...] = jnp.dot(hb, w1b_ref[...], preferred_element_type=f32)


def _prep_tables(h, w1a, w1b):
    bn = 1000
    return pl.pallas_call(
        _prep_body,
        grid=(N // bn,),
        in_specs=[
            pl.BlockSpec((bn, 128), lambda i: (i, 0)),
            pl.BlockSpec((128, 128), lambda i: (0, 0)),
            pl.BlockSpec((128, 128), lambda i: (0, 0)),
        ],
        out_specs=[
            pl.BlockSpec((bn, 128), lambda i: (i, 0)),
            pl.BlockSpec((bn, 128), lambda i: (i, 0)),
        ],
        out_shape=[
            jax.ShapeDtypeStruct((N, 128), f32),
            jax.ShapeDtypeStruct((N, 128), f32),
        ],
    )(h, w1a, w1b)


@functools.cache
def _gather_pre(e0, ne):
    PER_W = ne // NW
    CH = PER_W // C

    def body(ta, tb, row, col, out, idxr, idxc, bufr, bufc,
             semr0, semr1, semc0, semc1):
        c = lax.axis_index("c")
        s = lax.axis_index("s")
        wid = s * NC + c
        gbase = e0 + wid * PER_W
        obase = wid * PER_W
        pltpu.sync_copy(row.at[pl.ds(gbase, PER_W)], idxr)
        pltpu.sync_copy(col.at[pl.ds(gbase, PER_W)], idxc)
        semr = (semr0, semr1)
        semc = (semc0, semc1)

        def start(k, p):
            pltpu.async_copy(ta.at[idxr.at[pl.ds(k * C, C)]], bufr.at[p],
                             semr[p])
            pltpu.async_copy(tb.at[idxc.at[pl.ds(k * C, C)]], bufc.at[p],
                             semc[p])

        def bodyc(k, p):
            pltpu.make_async_copy(ta.at[idxr.at[pl.ds(0, C)]], bufr.at[p],
                                  semr[p]).wait()
            pltpu.make_async_copy(tb.at[idxc.at[pl.ds(0, C)]], bufc.at[p],
                                  semc[p]).wait()

            def rowfn(i, carry2):
                for j in range(8):
                    sl = pl.ds(j * 16, 16)
                    bufr[p, i, sl] = bufr[p, i, sl] + bufc[p, i, sl]
                return carry2

            lax.fori_loop(0, C, rowfn, 0)
            pltpu.sync_copy(bufr.at[p], out.at[pl.ds(obase + k * C, C)])

        start(0, 0)
        start(1, 1)

        def chunk2(k2, carry):
            for p in range(2):
                k = k2 * 2 + p
                bodyc(k, p)

                @pl.when(k + 2 < CH)
                def _():
                    start(k + 2, p)
            return carry

        lax.fori_loop(0, CH // 2, chunk2, 0)

        @pl.when((CH % 2) == 1)
        def _():
            bodyc(CH - 1, 0)

    return pl.kernel(
        body,
        out_type=jax.ShapeDtypeStruct((ne, 128), f32),
        mesh=plsc.VectorSubcoreMesh(**MESH),
        scratch_types=[
            pltpu.VMEM((PER_W,), i32),
            pltpu.VMEM((PER_W,), i32),
            pltpu.VMEM((2, C, 128), f32),
            pltpu.VMEM((2, C, 128), f32),
            pltpu.SemaphoreType.DMA,
            pltpu.SemaphoreType.DMA,
            pltpu.SemaphoreType.DMA,
            pltpu.SemaphoreType.DMA,
        ],
    )


@functools.cache
def _coord_feats(e0, ne):
    PER_W = ne // NW
    CH = PER_W // C

    def body(c16, row, col, out, idxr, idxc, bufr, bufc,
             semr0, semr1, semc0, semc1):
        c = lax.axis_index("c")
        s = lax.axis_index("s")
        wid = s * NC + c
        gbase = e0 + wid * PER_W
        obase = wid * PER_W
        pltpu.sync_copy(row.at[pl.ds(gbase, PER_W)], idxr)
        pltpu.sync_copy(col.at[pl.ds(gbase, PER_W)], idxc)
        semr = (semr0, semr1)
        semc = (semc0, semc1)

        def start(k, p):
            pltpu.async_copy(c16.at[idxr.at[pl.ds(k * C, C)]], bufr.at[p],
                             semr[p])
            pltpu.async_copy(c16.at[idxc.at[pl.ds(k * C, C)]], bufc.at[p],
                             semc[p])

        def bodyc(k, p):
            pltpu.make_async_copy(c16.at[idxr.at[pl.ds(0, C)]], bufr.at[p],
                                  semr[p]).wait()
            pltpu.make_async_copy(c16.at[idxc.at[pl.ds(0, C)]], bufc.at[p],
                                  semc[p]).wait()

            def rowfn(i, carry2):
                bufr[p, i, :] = bufr[p, i, :] - bufc[p, i, :]
                return carry2

            lax.fori_loop(0, C, rowfn, 0)
            pltpu.sync_copy(bufr.at[p], out.at[pl.ds(obase + k * C, C)])

        start(0, 0)
        start(1, 1)

        def chunk2(k2, carry):
            for p in range(2):
                k = k2 * 2 + p
                bodyc(k, p)

                @pl.when(k + 2 < CH)
                def _():
                    start(k + 2, p)
            return carry

        lax.fori_loop(0, CH // 2, chunk2, 0)

        @pl.when((CH % 2) == 1)
        def _():
            bodyc(CH - 1, 0)

    return pl.kernel(
        body,
        out_type=jax.ShapeDtypeStruct((ne, 16), f32),
        mesh=plsc.VectorSubcoreMesh(**MESH),
        compiler_params=pltpu.CompilerParams(use_tc_tiling_on_sc=False,
                                             needs_layout_passes=False),
        scratch_types=[
            pltpu.VMEM((PER_W,), i32),
            pltpu.VMEM((PER_W,), i32),
            pltpu.VMEM((2, C, 16), f32),
            pltpu.VMEM((2, C, 16), f32),
            pltpu.SemaphoreType.DMA,
            pltpu.SemaphoreType.DMA,
            pltpu.SemaphoreType.DMA,
            pltpu.SemaphoreType.DMA,
        ],
    )


def _edge_body(pre_ref, dr_ref, ea_ref, wr_ref, wea_ref, be1_ref,
               we2_ref, be2_ref, wc1_ref, bc1_ref, wc2_ref, bc2_ref,
               m_ref, t_ref):
    be = m_ref.shape[0]
    dr = dr_ref[...]
    rad = jnp.sum(dr * dr, axis=1, keepdims=True)
    x1 = (pre_ref[...] + rad * wr_ref[...] +
          jnp.dot(ea_ref[...], wea_ref[...], preferred_element_type=f32) +
          be1_ref[...])
    x1 = jnp.maximum(x1, 0.0)
    m = jnp.maximum(
        jnp.dot(x1, we2_ref[...], preferred_element_type=f32) + be2_ref[...],
        0.0)
    m_ref[...] = m
    cfh = jnp.maximum(
        jnp.dot(m, wc1_ref[...], preferred_element_type=f32) + bc1_ref[...],
        0.0)
    cf = jnp.dot(cfh, wc2_ref[...], preferred_element_type=f32) + bc2_ref[...]
    t = dr * cf
    iot = lax.broadcasted_iota(i32, t.shape, 1)
    t16 = jnp.where(iot == 3, 1.0, t)
    t_ref[...] = jnp.concatenate([t16, jnp.zeros((be, 112), f32)], axis=1)


def _edge_mlp(e0, ne, pre, dr, edge_attr,
              wr, wea, be1, we2, be2, wc1, bc1, wc2, bc2):
    be = 1600
    off = e0 // be
    wfull = lambda shape: pl.BlockSpec(shape, lambda i: (0, 0))
    return pl.pallas_call(
        _edge_body,
        grid=(ne // be,),
        in_specs=[
            pl.BlockSpec((be, 128), lambda i: (i + off, 0)),
            pl.BlockSpec((be, 16), lambda i: (i + off, 0)),
            pl.BlockSpec((be, DE), lambda i: (i + off, 0)),
            wfull((1, 128)), wfull((DE, 128)), wfull((1, 128)),
            wfull((128, 128)), wfull((1, 128)),
            wfull((128, 128)), wfull((1, 128)),
            wfull((128, 1)), wfull((1, 1)),
        ],
        out_specs=[
            pl.BlockSpec((be, 128), lambda i: (i, 0)),
            pl.BlockSpec((be, 128), lambda i: (i, 0)),
        ],
        out_shape=[
            jax.ShapeDtypeStruct((ne, 128), f32),
            jax.ShapeDtypeStruct((ne, 128), f32),
        ],
    )(pre, dr, edge_attr, wr, wea, be1, we2, be2, wc1, bc1, wc2, bc2)


def _make_scatter(e0, ne):
    PER_W = ne // NW
    CH = PER_W // C

    def body(val, row, z128, acc_out, idxv, vbuf, accsh, sem0, sem1):
        c = lax.axis_index("c")
        s = lax.axis_index("s")
        wid = s * NC + c
        ibase = e0 + wid * PER_W
        vbase = wid * PER_W
        rsl = pl.ds(s * ROWS_PER_TILE, ROWS_PER_TILE)
        pltpu.sync_copy(z128.at[rsl], accsh.at[rsl])
        plsc.subcore_barrier()
        sems = (sem0, sem1)

        def start(k, p):
            pltpu.sync_copy(row.at[pl.ds(ibase + k * C, C)], idxv.at[p])
            pltpu.async_copy(val.at[pl.ds(vbase + k * C, C)], vbuf.at[p],
                             sems[p])

        def bodyc(p):
            pltpu.make_async_copy(val.at[pl.ds(0, C)], vbuf.at[p],
                                  sems[p]).wait()
            pltpu.sync_copy(vbuf.at[p], accsh.at[idxv.at[p]], add=True)

        start(0, 0)
        start(1, 1)

        def chunk2(k2, carry):
            for p in range(2):
                k = k2 * 2 + p
                bodyc(p)

                @pl.when(k + 2 < CH)
                def _():
                    start(k + 2, p)
            return carry

        lax.fori_loop(0, CH // 2, chunk2, 0)

        @pl.when((CH % 2) == 1)
        def _():
            bodyc(0)

        plsc.subcore_barrier()
        pltpu.sync_copy(accsh.at[rsl], acc_out.at[c].at[rsl])

    return pl.kernel(
        body,
        out_type=jax.ShapeDtypeStruct((NC, NPAD, 128), f32),
        mesh=plsc.VectorSubcoreMesh(**MESH),
        scratch_types=[
            pltpu.VMEM((2, C), i32),
            pltpu.VMEM((2, C, 128), f32),
            pltpu.VMEM_SHARED((NPAD, 128), f32),
            pltpu.SemaphoreType.DMA,
            pltpu.SemaphoreType.DMA,
        ],
    )


@functools.cache
def _scatter_m(e0, ne):
    return _make_scatter(e0, ne)


@functools.cache
def _scatter_t(e0, ne):
    return _make_scatter(e0, ne)


def _node_body(h_ref, a00_ref, a01_ref, a10_ref, a11_ref,
               t00_ref, t01_ref, t10_ref, t11_ref, coord_ref,
               wn1a_ref, wn1b_ref, bn1_ref, wn2_ref, bn2_ref,
               hout_ref, cout_ref):
    agg = a00_ref[0] + a01_ref[0] + a10_ref[0] + a11_ref[0]
    u = jnp.maximum(
        jnp.dot(h_ref[...], wn1a_ref[...], preferred_element_type=f32) +
        jnp.dot(agg, wn1b_ref[...], preferred_element_type=f32) +
        bn1_ref[...], 0.0)
    hout_ref[...] = (jnp.dot(u, wn2_ref[...], preferred_element_type=f32) +
                     bn2_ref[...])
    t = t00_ref[0] + t01_ref[0] + t10_ref[0] + t11_ref[0]
    s3 = t[:, 0:3]
    cnt = t[:, 3:4]
    cout_ref[...] = coord_ref[...] + s3 / jnp.maximum(cnt, 1.0)


def _node_model(h, agg0, agg1, t0, t1, coord, wn1a, wn1b, bn1, wn2, bn2):
    bn = 1000
    wfull = lambda shape: pl.BlockSpec(shape, lambda i: (0, 0))
    p0 = pl.BlockSpec((1, bn, 128), lambda i: (0, i, 0))
    p1 = pl.BlockSpec((1, bn, 128), lambda i: (1, i, 0))
    return pl.pallas_call(
        _node_body,
        grid=(N // bn,),
        in_specs=[
            pl.BlockSpec((bn, 128), lambda i: (i, 0)),
            p0, p1, p0, p1, p0, p1, p0, p1,
            pl.BlockSpec((bn, 3), lambda i: (i, 0)),
            wfull((128, 128)), wfull((128, 128)), wfull((1, 128)),
            wfull((128, 128)), wfull((1, 128)),
        ],
        out_specs=[
            pl.BlockSpec((bn, 128), lambda i: (i, 0)),
            pl.BlockSpec((bn, 3), lambda i: (i, 0)),
        ],
        out_shape=[
            jax.ShapeDtypeStruct((N, 128), f32),
            jax.ShapeDtypeStruct((N, 3), f32),
        ],
    )(h, agg0, agg0, agg1, agg1, t0, t0, t1, t1, coord,
      wn1a, wn1b, bn1, wn2, bn2)


def kernel(h, edge_index, coord, edge_attr,
           W_e1, b_e1, W_e2, b_e2,
           W_n1, b_n1, W_n2, b_n2,
           W_c1, b_c1, W_c2, b_c2):
    row = edge_index[0]
    col = edge_index[1]
    c16 = jnp.pad(coord, ((0, 0), (0, 13)))
    w1a = W_e1[0:D]
    w1b = W_e1[D:2 * D]
    wr = W_e1[2 * D:2 * D + 1]
    wea = W_e1[2 * D + 1:]
    ew = (wr, wea, b_e1.reshape(1, H), W_e2, b_e2.reshape(1, H),
          W_c1, b_c1.reshape(1, H), W_c2, b_c2.reshape(1, 1))
    ta, tb = _prep_tables(h, w1a, w1b)
    z128 = jnp.zeros((NPAD, 128), f32)

    (a0, na), (a1, nb) = HALVES
    dr = _coord_feats(0, E)(c16, row, col)
    pre = _gather_pre(0, E)(ta, tb, row, col)
    m0, t0 = _edge_mlp(a0, na, pre, dr, edge_attr, *ew)
    m1, t1 = _edge_mlp(a1, nb, pre, dr, edge_attr, *ew)

    agg0 = _scatter_m(a0, na)(m0, row, z128)
    t_p0 = _scatter_t(a0, na)(t0, row, z128)
    agg1 = _scatter_m(a1, nb)(m1, row, z128)
    t_p1 = _scatter_t(a1, nb)(t1, row, z128)

    m_ij = jnp.concatenate([m0, m1], axis=0)
    h_out, coord_out = _node_model(h, agg0, agg1, t_p0, t_p1, coord,
                                   W_n1[0:D], W_n1[D:], b_n1.reshape(1, H),
                                   W_n2, b_n2.reshape(1, H))
    return (h_out, coord_out, m_ij)
